# baseline (speedup 1.0000x reference)
import functools

import jax
import jax.numpy as jnp
from jax import lax
from jax.experimental import pallas as pl
from jax.experimental.pallas import tpu as pltpu
from jax.experimental.pallas import tpu_sc as plsc

_NC = 2
_NS = 16
_NW = _NC * _NS

_CHUNK = 512


def _sc_gather_pairs(table128, idx_half):
    n, = idx_half.shape
    d = table128.shape[1]
    b_per_w = n // _NW
    n_chunks = b_per_w // _CHUNK
    mesh = plsc.VectorSubcoreMesh(core_axis_name="c", subcore_axis_name="s")

    @functools.partial(
        pl.kernel,
        mesh=mesh,
        out_type=jax.ShapeDtypeStruct((n, d), jnp.float32),
        scratch_types=[
            pltpu.VMEM((b_per_w,), jnp.int32),
            pltpu.VMEM((_CHUNK, d), jnp.float32),
            pltpu.SemaphoreType.DMA,
        ],
    )
    def gather_kernel(table_hbm, idx_hbm, out_hbm, idx_v, rows_v, sem):
        wid = lax.axis_index("s") * _NC + lax.axis_index("c")
        base = wid * b_per_w
        pltpu.sync_copy(idx_hbm.at[pl.ds(base, b_per_w)], idx_v)

        @pl.loop(0, n_chunks)
        def _(c):
            off = c * _CHUNK
            pltpu.async_copy(
                table_hbm.at[idx_v.at[pl.ds(off, _CHUNK)]], rows_v, sem
            ).wait()
            pltpu.sync_copy(rows_v, out_hbm.at[pl.ds(base + off, _CHUNK)])

    return gather_kernel(table128, idx_half)


def _mlp_block(rows_ref, par_ref, w1_ref, b1_ref, w2_ref, b2_ref, o_ref):
    window = rows_ref.shape[0]
    embed = rows_ref.shape[2] // 2
    h = b1_ref[...]
    for w in range(window):
        rw = rows_ref[w]
        p = par_ref[w][:, None]
        sel = jnp.where(p == 1, rw[:, embed:], rw[:, :embed])
        h = h + jnp.dot(sel, w1_ref[w], preferred_element_type=jnp.float32)
    h = jnp.maximum(h, 0.0)
    o_ref[...] = (
        jnp.dot(h, w2_ref[...], preferred_element_type=jnp.float32) + b2_ref[...]
    )


def _tc_mlp(rows, parity, w1s, b1, w2, b2, block_b=1024):
    window, batch, d2 = rows.shape
    embed = d2 // 2
    hidden = w2.shape[0]
    out_dim = w2.shape[1]
    return pl.pallas_call(
        _mlp_block,
        grid=(batch // block_b,),
        in_specs=[
            pl.BlockSpec((window, block_b, d2), lambda i: (0, i, 0)),
            pl.BlockSpec((window, block_b), lambda i: (0, i)),
            pl.BlockSpec((window, embed, hidden), lambda i: (0, 0, 0)),
            pl.BlockSpec((1, hidden), lambda i: (0, 0)),
            pl.BlockSpec((hidden, out_dim), lambda i: (0, 0)),
            pl.BlockSpec((1, out_dim), lambda i: (0, 0)),
        ],
        out_specs=pl.BlockSpec((block_b, out_dim), lambda i: (i, 0)),
        out_shape=jax.ShapeDtypeStruct((batch, out_dim), jnp.float32),
    )(rows, parity, w1s, b1, w2, b2)


def kernel(word_indices, table, W1, b1, W2, b2):
    batch, window = word_indices.shape
    vocab, embed = table.shape
    table128 = table.reshape(vocab // 2, 2 * embed)

    idx_wmajor = word_indices.T.reshape(-1).astype(jnp.int32)
    idx_half = idx_wmajor >> 1
    parity = (idx_wmajor & 1).reshape(window, batch)

    rows = _sc_gather_pairs(table128, idx_half)
    rows = rows.reshape(window, batch, 2 * embed)

    w1s = W1.reshape(window, embed, -1)
    out = _tc_mlp(rows, parity, w1s, b1.reshape(1, -1), W2, b2.reshape(1, -1))
    return out
